# baseline (speedup 1.0000x reference)
import functools

import jax
import jax.numpy as jnp
from jax import lax
from jax.experimental import pallas as pl
from jax.experimental.pallas import tpu as pltpu
from jax.experimental.pallas import tpu_sc as plsc

N_EDGES = 320000
D = 128
NUM_TYPES = 4
NC = 2
NS = 16
NW = NC * NS
CHUNK = 80
NBUF = 5
LOOK = 3

N_SC = 153600
N_TC = N_EDGES - N_SC
TC_BLOCK = 800

_mesh = plsc.VectorSubcoreMesh(core_axis_name="c", subcore_axis_name="s")


def _make_sc(n_edges):
    b_per_w = n_edges // NW
    nchunk = b_per_w // CHUNK
    ngrp = nchunk // NBUF
    assert nchunk * CHUNK == b_per_w and ngrp * NBUF == nchunk

    @functools.partial(
        pl.kernel,
        mesh=_mesh,
        out_type=jax.ShapeDtypeStruct((n_edges, D), jnp.float32),
        scratch_types=(
            [pltpu.VMEM((nchunk, CHUNK), jnp.int32),
             pltpu.VMEM_SHARED((NUM_TYPES, D), jnp.float32),
             pltpu.VMEM((NBUF, CHUNK, D), jnp.float32)]
            + [pltpu.SemaphoreType.DMA] * (2 * NBUF)
        ),
    )
    def _emb_lookup(idx_hbm, table_hbm, out_hbm, idx_v, table_v, rows, *sems):
        gsem = sems[:NBUF]
        wsem = sems[NBUF:]
        sid = lax.axis_index("s")
        wid = sid * NC + lax.axis_index("c")
        base = wid * b_per_w
        pltpu.sync_copy(idx_hbm.at[wid], idx_v)

        @pl.when(sid == 0)
        def _():
            pltpu.sync_copy(table_hbm, table_v)

        plsc.subcore_barrier()

        def gather(j, b):
            pltpu.async_copy(table_v.at[idx_v.at[j]], rows.at[b], gsem[b])

        def gather_wait(j, b):
            pltpu.make_async_copy(
                table_v.at[idx_v.at[j]], rows.at[b], gsem[b]).wait()

        def write(j, b):
            pltpu.async_copy(
                rows.at[b], out_hbm.at[pl.ds(base + j * CHUNK, CHUNK), :],
                wsem[b])

        def write_wait(b):
            pltpu.make_async_copy(
                rows.at[b], out_hbm.at[pl.ds(base, CHUNK), :], wsem[b]).wait()

        for j in range(LOOK):
            gather(j, j % NBUF)

        for db in range(NBUF):
            j, b = db, db
            gather_wait(j, b)
            write(j, b)
            jn, bn = j + LOOK, (db + LOOK) % NBUF
            if jn >= NBUF:
                write_wait(bn)
            gather(jn, bn)

        def group(g, carry):
            j0 = g * NBUF
            for db in range(NBUF):
                j, b = j0 + db, db
                gather_wait(j, b)
                write(j, b)
                jn, bn = j + LOOK, (db + LOOK) % NBUF
                write_wait(bn)
                gather(jn, bn)
            return carry

        lax.fori_loop(1, ngrp - 1, group, 0)

        for db in range(NBUF):
            j, b = (ngrp - 1) * NBUF + db, db
            gather_wait(j, b)
            write(j, b)
            jn, bn = j + LOOK, (db + LOOK) % NBUF
            if jn < nchunk:
                write_wait(bn)
                gather(jn, bn)

        for b in range(NBUF):
            write_wait(b)

    return _emb_lookup


_sc_lookup = _make_sc(N_SC)


def _tc_body(idx_ref, table_ref, out_ref):
    idx = idx_ref[0, 0, :]
    onehot = (idx[:, None] == lax.broadcasted_iota(jnp.int32, (1, NUM_TYPES), 1)
              ).astype(jnp.float32)
    out_ref[...] = jnp.dot(onehot, table_ref[...],
                           preferred_element_type=jnp.float32)


_tc_lookup = pl.pallas_call(
    _tc_body,
    grid=(N_TC // TC_BLOCK,),
    in_specs=[
        pl.BlockSpec((1, 1, TC_BLOCK), lambda i: (i, 0, 0)),
        pl.BlockSpec((NUM_TYPES, D), lambda i: (0, 0)),
    ],
    out_specs=pl.BlockSpec((TC_BLOCK, D), lambda i: (i, 0)),
    out_shape=jax.ShapeDtypeStruct((N_TC, D), jnp.float32),
)


def kernel(edge_attr, emb_weight):
    idx_sc = edge_attr[:N_SC].reshape(NW, N_SC // NW // CHUNK, CHUNK)
    idx_tc = edge_attr[N_SC:].reshape(N_TC // TC_BLOCK, 1, TC_BLOCK)
    out_sc = _sc_lookup(idx_sc, emb_weight)
    out_tc = _tc_lookup(idx_tc, emb_weight)
    return jnp.concatenate([out_sc, out_tc], axis=0)
